# Initial kernel scaffold; baseline (speedup 1.0000x reference)
import functools

import jax
import jax.numpy as jnp
from jax import lax
from jax.experimental import pallas as pl
from jax.experimental.pallas import tpu as pltpu
from jax.experimental.pallas import tpu_sc as plsc

N = 10000
E = 320000
D = 128
A = 16

NC = 2
NS = 16
NW = NC * NS

CHUNK = 80
EPW = E // NW
NCH = EPW // CHUNK
RPT = N // NS

_SC_MESH = plsc.VectorSubcoreMesh(core_axis_name="c", subcore_axis_name="s")


@functools.partial(
    pl.kernel,
    out_type=jax.ShapeDtypeStruct((E, D), jnp.float32),
    mesh=_SC_MESH,
    scratch_types=[
        pltpu.VMEM((NCH, CHUNK), jnp.int32),
        pltpu.VMEM((CHUNK, D), jnp.float32),
        pltpu.SemaphoreType.DMA,
    ],
)
def _sc_gather(table_hbm, idx_hbm, out_hbm, idx_v, rows_v, sem):
    wid = lax.axis_index("s") * NC + lax.axis_index("c")
    ebase = wid * EPW
    pltpu.sync_copy(idx_hbm.at[pl.ds(wid * NCH, NCH)], idx_v)

    @pl.loop(0, NCH)
    def _(j):
        pltpu.async_copy(table_hbm.at[idx_v.at[j]], rows_v, sem).wait()
        pltpu.sync_copy(rows_v, out_hbm.at[pl.ds(ebase + j * CHUNK, CHUNK)])


@functools.partial(
    pl.kernel,
    out_type=jax.ShapeDtypeStruct((NC, N, D), jnp.float32),
    mesh=_SC_MESH,
    scratch_types=[
        pltpu.VMEM((NCH, CHUNK), jnp.int32),
        pltpu.VMEM((CHUNK, D), jnp.float32),
        pltpu.VMEM_SHARED((N, D), jnp.float32),
    ],
)
def _sc_scatter(m_hbm, idx_hbm, zeros_hbm, out_hbm, idx_v, rows_v, acc):
    c = lax.axis_index("c")
    s = lax.axis_index("s")
    ebase = c * (E // 2) + s * EPW
    cbase = c * (E // 2 // CHUNK) + s * NCH
    pltpu.sync_copy(idx_hbm.at[pl.ds(cbase, NCH)], idx_v)
    pltpu.sync_copy(zeros_hbm, acc.at[pl.ds(s * RPT, RPT)])
    plsc.subcore_barrier()

    @pl.loop(0, NCH)
    def _(j):
        pltpu.sync_copy(m_hbm.at[pl.ds(ebase + j * CHUNK, CHUNK)], rows_v)
        pltpu.sync_copy(rows_v, acc.at[idx_v.at[j]], add=True)

    plsc.subcore_barrier()
    pltpu.sync_copy(acc.at[pl.ds(s * RPT, RPT)], out_hbm.at[c, pl.ds(s * RPT, RPT)])


def _p_body(x_ref, w_ref, b_ref, o_ref):
    o_ref[...] = (
        jnp.dot(x_ref[...], w_ref[...], preferred_element_type=jnp.float32)
        + b_ref[...]
    )


def _mlp_body(g_ref, e_ref, w1e_ref, w2_ref, b2_ref, o_ref):
    h = g_ref[...] + jnp.dot(
        e_ref[...], w1e_ref[...], preferred_element_type=jnp.float32
    )
    h = h * jax.nn.sigmoid(h)
    m = jnp.dot(h, w2_ref[...], preferred_element_type=jnp.float32) + b2_ref[...]
    o_ref[...] = m * jax.nn.sigmoid(m)


def _gru_body(a2_ref, x_ref, wih_ref, whh_ref, bih_ref, bhh_ref, o_ref):
    agg = a2_ref[0] + a2_ref[1]
    h = x_ref[...]
    gi = jnp.dot(agg, wih_ref[...], preferred_element_type=jnp.float32) + bih_ref[...]
    gh = jnp.dot(h, whh_ref[...], preferred_element_type=jnp.float32) + bhh_ref[...]
    r = jax.nn.sigmoid(gi[:, :D] + gh[:, :D])
    z = jax.nn.sigmoid(gi[:, D : 2 * D] + gh[:, D : 2 * D])
    n = jnp.tanh(gi[:, 2 * D :] + r * gh[:, 2 * D :])
    o_ref[...] = (1.0 - z) * n + z * h


EB = 3200
NB = 2000


def kernel(x, edge_index, edge_attr, W1, b1, W2, b2, W_ih, W_hh, b_ih, b_hh):
    src2d = edge_index[0].reshape(E // CHUNK, CHUNK)
    dst2d = edge_index[1].reshape(E // CHUNK, CHUNK)
    w1x_t = W1[:, :D].T
    w1e_t = W1[:, D:].T
    w2_t = W2.T
    wih_t = W_ih.T
    whh_t = W_hh.T

    p = pl.pallas_call(
        _p_body,
        grid=(N // NB,),
        in_specs=[
            pl.BlockSpec((NB, D), lambda i: (i, 0)),
            pl.BlockSpec((D, D), lambda i: (0, 0)),
            pl.BlockSpec((1, D), lambda i: (0, 0)),
        ],
        out_specs=pl.BlockSpec((NB, D), lambda i: (i, 0)),
        out_shape=jax.ShapeDtypeStruct((N, D), jnp.float32),
    )(x, w1x_t, b1.reshape(1, D))

    g = _sc_gather(p, src2d)

    m = pl.pallas_call(
        _mlp_body,
        grid=(E // EB,),
        in_specs=[
            pl.BlockSpec((EB, D), lambda i: (i, 0)),
            pl.BlockSpec((EB, A), lambda i: (i, 0)),
            pl.BlockSpec((A, D), lambda i: (0, 0)),
            pl.BlockSpec((D, D), lambda i: (0, 0)),
            pl.BlockSpec((1, D), lambda i: (0, 0)),
        ],
        out_specs=pl.BlockSpec((EB, D), lambda i: (i, 0)),
        out_shape=jax.ShapeDtypeStruct((E, D), jnp.float32),
    )(g, edge_attr, w1e_t, w2_t, b2.reshape(1, D))

    agg2 = _sc_scatter(m, dst2d, jnp.zeros((RPT, D), jnp.float32))

    out = pl.pallas_call(
        _gru_body,
        grid=(N // NB,),
        in_specs=[
            pl.BlockSpec((NC, NB, D), lambda i: (0, i, 0)),
            pl.BlockSpec((NB, D), lambda i: (i, 0)),
            pl.BlockSpec((D, 3 * D), lambda i: (0, 0)),
            pl.BlockSpec((D, 3 * D), lambda i: (0, 0)),
            pl.BlockSpec((1, 3 * D), lambda i: (0, 0)),
            pl.BlockSpec((1, 3 * D), lambda i: (0, 0)),
        ],
        out_specs=pl.BlockSpec((NB, D), lambda i: (i, 0)),
        out_shape=jax.ShapeDtypeStruct((N, D), jnp.float32),
    )(agg2, x, wih_t, whh_t, b_ih.reshape(1, 3 * D), b_hh.reshape(1, 3 * D))

    return out


# speedup vs baseline: 3.0810x; 3.0810x over previous
import functools

import jax
import jax.numpy as jnp
from jax import lax
from jax.experimental import pallas as pl
from jax.experimental.pallas import tpu as pltpu
from jax.experimental.pallas import tpu_sc as plsc

N = 10000
E = 320000
D = 128
A = 16

NC = 2
NS = 16
NW = NC * NS

CHUNK = 80
EPW = E // NW
NCH = EPW // CHUNK
RPT = 624
REM = N - NS * RPT

_SC_MESH = plsc.VectorSubcoreMesh(core_axis_name="c", subcore_axis_name="s")


@functools.partial(
    pl.kernel,
    out_type=jax.ShapeDtypeStruct((E, D), jnp.float32),
    mesh=_SC_MESH,
    scratch_types=[
        pltpu.VMEM((NCH, CHUNK), jnp.int32),
        pltpu.VMEM((CHUNK, D), jnp.float32),
        pltpu.SemaphoreType.DMA,
    ],
)
def _sc_gather(table_hbm, idx_hbm, out_hbm, idx_v, rows_v, sem):
    wid = lax.axis_index("c") * NS + lax.axis_index("s")
    ebase = wid * EPW
    pltpu.sync_copy(idx_hbm.at[wid], idx_v)

    @pl.loop(0, NCH)
    def _(j):
        pltpu.async_copy(table_hbm.at[idx_v.at[j]], rows_v, sem).wait()
        pltpu.sync_copy(rows_v, out_hbm.at[pl.ds(ebase + j * CHUNK, CHUNK)])


@functools.partial(
    pl.kernel,
    out_type=jax.ShapeDtypeStruct((NC, N, D), jnp.float32),
    mesh=_SC_MESH,
    scratch_types=[
        pltpu.VMEM((NCH, CHUNK), jnp.int32),
        pltpu.VMEM((CHUNK, D), jnp.float32),
        pltpu.VMEM_SHARED((N, D), jnp.float32),
    ],
)
def _sc_scatter(m_hbm, idx_hbm, zeros_hbm, out_hbm, idx_v, rows_v, acc):
    c = lax.axis_index("c")
    s = lax.axis_index("s")
    wid = c * NS + s
    ebase = wid * EPW
    pltpu.sync_copy(idx_hbm.at[wid], idx_v)
    pltpu.sync_copy(zeros_hbm, acc.at[pl.ds(s * RPT, RPT)])

    @pl.when(s == 0)
    def _():
        pltpu.sync_copy(zeros_hbm.at[pl.ds(0, REM)], acc.at[pl.ds(NS * RPT, REM)])

    plsc.subcore_barrier()

    @pl.loop(0, NCH)
    def _(j):
        pltpu.sync_copy(m_hbm.at[pl.ds(ebase + j * CHUNK, CHUNK)], rows_v)
        pltpu.sync_copy(rows_v, acc.at[idx_v.at[j]], add=True)

    plsc.subcore_barrier()
    pltpu.sync_copy(acc.at[pl.ds(s * RPT, RPT)], out_hbm.at[c, pl.ds(s * RPT, RPT)])

    @pl.when(s == 0)
    def _():
        pltpu.sync_copy(
            acc.at[pl.ds(NS * RPT, REM)], out_hbm.at[c, pl.ds(NS * RPT, REM)]
        )


def _p_body(x_ref, w_ref, b_ref, o_ref):
    o_ref[...] = (
        jnp.dot(x_ref[...], w_ref[...], preferred_element_type=jnp.float32)
        + b_ref[...]
    )


def _mlp_body(g_ref, e_ref, w1e_ref, w2_ref, b2_ref, o_ref):
    h = g_ref[...] + jnp.dot(
        e_ref[...], w1e_ref[...], preferred_element_type=jnp.float32
    )
    h = h * jax.nn.sigmoid(h)
    m = jnp.dot(h, w2_ref[...], preferred_element_type=jnp.float32) + b2_ref[...]
    o_ref[...] = m * jax.nn.sigmoid(m)


def _gru_body(a2_ref, x_ref, wih_ref, whh_ref, bih_ref, bhh_ref, o_ref):
    agg = a2_ref[0] + a2_ref[1]
    h = x_ref[...]
    gi = jnp.dot(agg, wih_ref[...], preferred_element_type=jnp.float32) + bih_ref[...]
    gh = jnp.dot(h, whh_ref[...], preferred_element_type=jnp.float32) + bhh_ref[...]
    r = jax.nn.sigmoid(gi[:, :D] + gh[:, :D])
    z = jax.nn.sigmoid(gi[:, D : 2 * D] + gh[:, D : 2 * D])
    n = jnp.tanh(gi[:, 2 * D :] + r * gh[:, 2 * D :])
    o_ref[...] = (1.0 - z) * n + z * h


EB = 3200
NB = 2000


def kernel(x, edge_index, edge_attr, W1, b1, W2, b2, W_ih, W_hh, b_ih, b_hh):
    src2d = edge_index[0].reshape(NW, NCH, CHUNK)
    dst2d = edge_index[1].reshape(NW, NCH, CHUNK)
    w1x_t = W1[:, :D].T
    w1e_t = W1[:, D:].T
    w2_t = W2.T
    wih_t = W_ih.T
    whh_t = W_hh.T

    p = pl.pallas_call(
        _p_body,
        grid=(N // NB,),
        in_specs=[
            pl.BlockSpec((NB, D), lambda i: (i, 0)),
            pl.BlockSpec((D, D), lambda i: (0, 0)),
            pl.BlockSpec((1, D), lambda i: (0, 0)),
        ],
        out_specs=pl.BlockSpec((NB, D), lambda i: (i, 0)),
        out_shape=jax.ShapeDtypeStruct((N, D), jnp.float32),
    )(x, w1x_t, b1.reshape(1, D))

    g = _sc_gather(p, src2d)

    m = pl.pallas_call(
        _mlp_body,
        grid=(E // EB,),
        in_specs=[
            pl.BlockSpec((EB, D), lambda i: (i, 0)),
            pl.BlockSpec((EB, A), lambda i: (i, 0)),
            pl.BlockSpec((A, D), lambda i: (0, 0)),
            pl.BlockSpec((D, D), lambda i: (0, 0)),
            pl.BlockSpec((1, D), lambda i: (0, 0)),
        ],
        out_specs=pl.BlockSpec((EB, D), lambda i: (i, 0)),
        out_shape=jax.ShapeDtypeStruct((E, D), jnp.float32),
    )(g, edge_attr, w1e_t, w2_t, b2.reshape(1, D))

    agg2 = _sc_scatter(m, dst2d, jnp.zeros((RPT, D), jnp.float32))

    out = pl.pallas_call(
        _gru_body,
        grid=(N // NB,),
        in_specs=[
            pl.BlockSpec((NC, NB, D), lambda i: (0, i, 0)),
            pl.BlockSpec((NB, D), lambda i: (i, 0)),
            pl.BlockSpec((D, 3 * D), lambda i: (0, 0)),
            pl.BlockSpec((D, 3 * D), lambda i: (0, 0)),
            pl.BlockSpec((1, 3 * D), lambda i: (0, 0)),
            pl.BlockSpec((1, 3 * D), lambda i: (0, 0)),
        ],
        out_specs=pl.BlockSpec((NB, D), lambda i: (i, 0)),
        out_shape=jax.ShapeDtypeStruct((N, D), jnp.float32),
    )(agg2, x, wih_t, whh_t, b_ih.reshape(1, 3 * D), b_hh.reshape(1, 3 * D))

    return out


# speedup vs baseline: 3.8813x; 1.2598x over previous
import functools

import jax
import jax.numpy as jnp
from jax import lax
from jax.experimental import pallas as pl
from jax.experimental.pallas import tpu as pltpu
from jax.experimental.pallas import tpu_sc as plsc

N = 10000
E = 320000
D = 128
A = 16

NC = 2
NS = 16
NW = NC * NS

CHUNK = 80
EPW = E // NW
NCH = EPW // CHUNK
RPT = 624
REM = N - NS * RPT

_SC_MESH = plsc.VectorSubcoreMesh(core_axis_name="c", subcore_axis_name="s")


@functools.partial(
    pl.kernel,
    out_type=jax.ShapeDtypeStruct((E, D), jnp.float32),
    mesh=_SC_MESH,
    scratch_types=[
        pltpu.VMEM((NCH, CHUNK), jnp.int32),
        pltpu.VMEM((CHUNK, D), jnp.float32),
        pltpu.VMEM((CHUNK, D), jnp.float32),
        pltpu.SemaphoreType.DMA,
        pltpu.SemaphoreType.DMA,
    ],
)
def _sc_gather(table_hbm, idx_hbm, out_hbm, idx_v, rows0, rows1, sem0, sem1):
    wid = lax.axis_index("c") * NS + lax.axis_index("s")
    ebase = wid * EPW
    pltpu.sync_copy(idx_hbm.at[wid], idx_v)

    def _gath(j, buf, sem):
        pltpu.async_copy(table_hbm.at[idx_v.at[j]], buf, sem)

    def _wait(j, buf, sem):
        pltpu.make_async_copy(table_hbm.at[idx_v.at[j]], buf, sem).wait()

    def _put(j, buf):
        pltpu.sync_copy(buf, out_hbm.at[pl.ds(ebase + j * CHUNK, CHUNK)])

    _gath(0, rows0, sem0)

    @pl.loop(0, NCH - 1, step=2)
    def _(j):
        _gath(j + 1, rows1, sem1)
        _wait(j, rows0, sem0)
        _put(j, rows0)
        _gath(j + 2, rows0, sem0)
        _wait(j + 1, rows1, sem1)
        _put(j + 1, rows1)

    _wait(NCH - 1, rows0, sem0)
    _put(NCH - 1, rows0)


@functools.partial(
    pl.kernel,
    out_type=jax.ShapeDtypeStruct((NC, N, D), jnp.float32),
    mesh=_SC_MESH,
    scratch_types=[
        pltpu.VMEM((NCH, CHUNK), jnp.int32),
        pltpu.VMEM((CHUNK, D), jnp.float32),
        pltpu.VMEM((CHUNK, D), jnp.float32),
        pltpu.VMEM_SHARED((N, D), jnp.float32),
        pltpu.SemaphoreType.DMA,
        pltpu.SemaphoreType.DMA,
    ],
)
def _sc_scatter(m_hbm, idx_hbm, zeros_hbm, out_hbm, idx_v, rows0, rows1, acc, sem0, sem1):
    c = lax.axis_index("c")
    s = lax.axis_index("s")
    wid = c * NS + s
    ebase = wid * EPW
    pltpu.sync_copy(idx_hbm.at[wid], idx_v)
    pltpu.sync_copy(zeros_hbm, acc.at[pl.ds(s * RPT, RPT)])

    @pl.when(s == 0)
    def _():
        pltpu.sync_copy(zeros_hbm.at[pl.ds(0, REM)], acc.at[pl.ds(NS * RPT, REM)])

    plsc.subcore_barrier()

    def _get(j, buf, sem):
        pltpu.async_copy(m_hbm.at[pl.ds(ebase + j * CHUNK, CHUNK)], buf, sem)

    def _wait(j, buf, sem):
        pltpu.make_async_copy(
            m_hbm.at[pl.ds(ebase + j * CHUNK, CHUNK)], buf, sem
        ).wait()

    def _acc(j, buf):
        pltpu.sync_copy(buf, acc.at[idx_v.at[j]], add=True)

    _get(0, rows0, sem0)

    @pl.loop(0, NCH - 1, step=2)
    def _(j):
        _get(j + 1, rows1, sem1)
        _wait(j, rows0, sem0)
        _acc(j, rows0)
        _get(j + 2, rows0, sem0)
        _wait(j + 1, rows1, sem1)
        _acc(j + 1, rows1)

    _wait(NCH - 1, rows0, sem0)
    _acc(NCH - 1, rows0)

    plsc.subcore_barrier()
    pltpu.sync_copy(acc.at[pl.ds(s * RPT, RPT)], out_hbm.at[c, pl.ds(s * RPT, RPT)])

    @pl.when(s == 0)
    def _():
        pltpu.sync_copy(
            acc.at[pl.ds(NS * RPT, REM)], out_hbm.at[c, pl.ds(NS * RPT, REM)]
        )


def _p_body(x_ref, w_ref, b_ref, o_ref):
    o_ref[...] = (
        jnp.dot(x_ref[...], w_ref[...], preferred_element_type=jnp.float32)
        + b_ref[...]
    )


def _mlp_body(g_ref, e_ref, w1e_ref, w2_ref, b2_ref, o_ref):
    h = g_ref[...] + jnp.dot(
        e_ref[...], w1e_ref[...], preferred_element_type=jnp.float32
    )
    h = h * jax.nn.sigmoid(h)
    m = jnp.dot(h, w2_ref[...], preferred_element_type=jnp.float32) + b2_ref[...]
    o_ref[...] = m * jax.nn.sigmoid(m)


def _gru_body(a2_ref, x_ref, wih_ref, whh_ref, bih_ref, bhh_ref, o_ref):
    agg = a2_ref[0] + a2_ref[1]
    h = x_ref[...]
    gi = jnp.dot(agg, wih_ref[...], preferred_element_type=jnp.float32) + bih_ref[...]
    gh = jnp.dot(h, whh_ref[...], preferred_element_type=jnp.float32) + bhh_ref[...]
    r = jax.nn.sigmoid(gi[:, :D] + gh[:, :D])
    z = jax.nn.sigmoid(gi[:, D : 2 * D] + gh[:, D : 2 * D])
    n = jnp.tanh(gi[:, 2 * D :] + r * gh[:, 2 * D :])
    o_ref[...] = (1.0 - z) * n + z * h


EB = 3200
NB = 2000


def kernel(x, edge_index, edge_attr, W1, b1, W2, b2, W_ih, W_hh, b_ih, b_hh):
    src2d = edge_index[0].reshape(NW, NCH, CHUNK)
    dst2d = edge_index[1].reshape(NW, NCH, CHUNK)
    w1x_t = W1[:, :D].T
    w1e_t = W1[:, D:].T
    w2_t = W2.T
    wih_t = W_ih.T
    whh_t = W_hh.T

    p = pl.pallas_call(
        _p_body,
        grid=(N // NB,),
        in_specs=[
            pl.BlockSpec((NB, D), lambda i: (i, 0)),
            pl.BlockSpec((D, D), lambda i: (0, 0)),
            pl.BlockSpec((1, D), lambda i: (0, 0)),
        ],
        out_specs=pl.BlockSpec((NB, D), lambda i: (i, 0)),
        out_shape=jax.ShapeDtypeStruct((N, D), jnp.float32),
    )(x, w1x_t, b1.reshape(1, D))

    g = _sc_gather(p, src2d)

    m = pl.pallas_call(
        _mlp_body,
        grid=(E // EB,),
        in_specs=[
            pl.BlockSpec((EB, D), lambda i: (i, 0)),
            pl.BlockSpec((EB, A), lambda i: (i, 0)),
            pl.BlockSpec((A, D), lambda i: (0, 0)),
            pl.BlockSpec((D, D), lambda i: (0, 0)),
            pl.BlockSpec((1, D), lambda i: (0, 0)),
        ],
        out_specs=pl.BlockSpec((EB, D), lambda i: (i, 0)),
        out_shape=jax.ShapeDtypeStruct((E, D), jnp.float32),
    )(g, edge_attr, w1e_t, w2_t, b2.reshape(1, D))

    agg2 = _sc_scatter(m, dst2d, jnp.zeros((RPT, D), jnp.float32))

    out = pl.pallas_call(
        _gru_body,
        grid=(N // NB,),
        in_specs=[
            pl.BlockSpec((NC, NB, D), lambda i: (0, i, 0)),
            pl.BlockSpec((NB, D), lambda i: (i, 0)),
            pl.BlockSpec((D, 3 * D), lambda i: (0, 0)),
            pl.BlockSpec((D, 3 * D), lambda i: (0, 0)),
            pl.BlockSpec((1, 3 * D), lambda i: (0, 0)),
            pl.BlockSpec((1, 3 * D), lambda i: (0, 0)),
        ],
        out_specs=pl.BlockSpec((NB, D), lambda i: (i, 0)),
        out_shape=jax.ShapeDtypeStruct((N, D), jnp.float32),
    )(agg2, x, wih_t, whh_t, b_ih.reshape(1, 3 * D), b_hh.reshape(1, 3 * D))

    return out
